# Initial kernel scaffold; baseline (speedup 1.0000x reference)
import jax
import jax.numpy as jnp
from jax.experimental import pallas as pl


def kernel(feat, edge_index, e_feat, att):
    raise NotImplementedError("write your pallas kernel here")


# speedup vs baseline: 228.1230x; 228.1230x over previous
import functools

import jax
import jax.numpy as jnp
from jax import lax
from jax.experimental import pallas as pl
from jax.experimental.pallas import tpu as pltpu
from jax.experimental.pallas import tpu_sc as plsc

B = 25
N_PER = 400
N = B * N_PER
DEG = 16
E = N * DEG
D = 128
K = 320
PN = B * K
KK = K * K
NC = B * KK
E_PER = N_PER * DEG
HALF = N_PER // 2
LAMB = 1.0
SLOPE = 0.2

_SC_PARAMS = pltpu.CompilerParams(needs_layout_passes=False)


def _sc_build_adj(src, dst, ef):
    mesh = plsc.VectorSubcoreMesh(core_axis_name="c", subcore_axis_name="s")

    @functools.partial(
        pl.kernel,
        mesh=mesh,
        out_type=(
            jax.ShapeDtypeStruct((B * N_PER * N_PER,), jnp.float32),
            jax.ShapeDtypeStruct((B * 2 * N_PER,), jnp.float32),
        ),
        scratch_types=[
            pltpu.VMEM((HALF * N_PER,), jnp.float32),
            pltpu.VMEM((E_PER,), jnp.int32),
            pltpu.VMEM((E_PER,), jnp.int32),
            pltpu.VMEM((E_PER,), jnp.float32),
            pltpu.VMEM((2 * N_PER,), jnp.float32),
        ],
        compiler_params=_SC_PARAMS,
    )
    def k(src_hbm, dst_hbm, ef_hbm, a_hbm, deg_hbm, a_buf, s_buf, d_buf, e_buf, deg_buf):
        c = lax.axis_index("c")
        s = lax.axis_index("s")
        wid = s * 2 + c
        zero16f = jnp.zeros((16,), jnp.float32)
        ones16 = jnp.ones((16,), jnp.float32)
        for it in range(2):
            item = wid + it * 32

            @pl.when(item < 2 * B)
            def _():
                g = item // 2
                half = item % 2
                pltpu.sync_copy(src_hbm.at[pl.ds(g * E_PER, E_PER)], s_buf)
                pltpu.sync_copy(dst_hbm.at[pl.ds(g * E_PER, E_PER)], d_buf)
                pltpu.sync_copy(ef_hbm.at[pl.ds(g * E_PER, E_PER)], e_buf)

                def zbody(i, _):
                    a_buf[pl.ds(i * 16, 16)] = zero16f
                    return 0

                lax.fori_loop(0, (HALF * N_PER) // 16, zbody, 0)

                @pl.when(half == 0)
                def _():
                    def zb2(i, _):
                        deg_buf[pl.ds(i * 16, 16)] = zero16f
                        return 0

                    lax.fori_loop(0, (2 * N_PER) // 16, zb2, 0)

                goff = g * N_PER
                dhalf = half * HALF

                def body(i, _):
                    sl = s_buf[pl.ds(i * 16, 16)] - goff
                    dl = d_buf[pl.ds(i * 16, 16)] - goff
                    ev = e_buf[pl.ds(i * 16, 16)]
                    dd = dl - dhalf
                    m = (dd >= 0) & (dd < HALF)
                    idx = jnp.where(m, dd * N_PER + sl, 0)
                    plsc.addupdate_scatter(a_buf, [idx], ev, mask=m)

                    @pl.when(half == 0)
                    def _():
                        plsc.addupdate_scatter(deg_buf, [sl], ones16)
                        plsc.addupdate_scatter(deg_buf, [dl + N_PER], ones16)

                    return 0

                lax.fori_loop(0, E_PER // 16, body, 0)
                pltpu.sync_copy(
                    a_buf,
                    a_hbm.at[pl.ds(g * N_PER * N_PER + half * (HALF * N_PER), HALF * N_PER)],
                )

                @pl.when(half == 0)
                def _():
                    pltpu.sync_copy(deg_buf, deg_hbm.at[pl.ds(g * 2 * N_PER, 2 * N_PER)])

    return k(src, dst, ef)


def _sc_build_bias(src, dst, ef, nm):
    mesh = plsc.VectorSubcoreMesh(core_axis_name="c", subcore_axis_name="s")

    @functools.partial(
        pl.kernel,
        mesh=mesh,
        out_type=jax.ShapeDtypeStruct((NC,), jnp.float32),
        scratch_types=[
            pltpu.VMEM((KK,), jnp.float32),
            pltpu.VMEM((E_PER,), jnp.int32),
            pltpu.VMEM((E_PER,), jnp.int32),
            pltpu.VMEM((E_PER,), jnp.float32),
            pltpu.VMEM((N_PER,), jnp.int32),
        ],
        compiler_params=_SC_PARAMS,
    )
    def k(src_hbm, dst_hbm, ef_hbm, nm_hbm, bias_hbm, bias_buf, s_buf, d_buf, e_buf, nm_buf):
        c = lax.axis_index("c")
        s = lax.axis_index("s")
        wid = s * 2 + c
        zero16f = jnp.zeros((16,), jnp.float32)

        @pl.when(wid < B)
        def _():
            g = wid
            pltpu.sync_copy(src_hbm.at[pl.ds(g * E_PER, E_PER)], s_buf)
            pltpu.sync_copy(dst_hbm.at[pl.ds(g * E_PER, E_PER)], d_buf)
            pltpu.sync_copy(ef_hbm.at[pl.ds(g * E_PER, E_PER)], e_buf)
            pltpu.sync_copy(nm_hbm.at[pl.ds(g * N_PER, N_PER)], nm_buf)

            def zbody(i, _):
                bias_buf[pl.ds(i * 16, 16)] = zero16f
                return 0

            lax.fori_loop(0, KK // 16, zbody, 0)

            goff = g * N_PER

            def body(i, _):
                sl = s_buf[pl.ds(i * 16, 16)] - goff
                dl = d_buf[pl.ds(i * 16, 16)] - goff
                ev = e_buf[pl.ds(i * 16, 16)]
                ms = plsc.load_gather(nm_buf, [sl])
                md = plsc.load_gather(nm_buf, [dl])
                valid = (ms >= 0) & (md >= 0)
                idx = jnp.where(valid, ms * K + md, 0)
                plsc.addupdate_scatter(bias_buf, [idx], ev, mask=valid)
                return 0

            lax.fori_loop(0, E_PER // 16, body, 0)
            pltpu.sync_copy(bias_buf, bias_hbm.at[pl.ds(g * KK, KK)])

    return k(src, dst, ef, nm)


def _dot(x, y, dims):
    return lax.dot_general(
        x, y, (dims, ((), ())),
        precision=lax.Precision.HIGHEST, preferred_element_type=jnp.float32,
    )


def _tc_score_topk(A, feat_r, src_n, dst_n, att2):
    def body(a_ref, f_ref, sn_ref, dn_ref, att_ref, fp_ref, perm_ref, nm_ref, av_ref, bv_ref):
        g = pl.program_id(0)
        Ag = a_ref[0]
        fg = f_ref[0]
        sn = sn_ref[0]
        dn = dn_ref[0]
        x = fg * sn
        agg = _dot(Ag, x, ((1,), (0,)))
        f2 = fg - agg * dn
        score_col = jnp.sum(jnp.abs(f2), axis=1, keepdims=True)
        ii = lax.broadcasted_iota(jnp.int32, (N_PER, N_PER), 0)
        jj = lax.broadcasted_iota(jnp.int32, (N_PER, N_PER), 1)
        eye_n = jnp.where(ii == jj, 1.0, 0.0)
        score_row = _dot(score_col, eye_n, ((0,), (0,)))
        gt = score_row > score_col
        eq = score_row == score_col
        cmp_f = jnp.where(gt | (eq & (jj < ii)), 1.0, 0.0)
        ones_col = jnp.ones((N_PER, 1), jnp.float32)
        rank_col = _dot(cmp_f, ones_col, ((1,), (0,)))
        rank_row = _dot(rank_col, eye_n, ((0,), (0,)))
        rank_i = rank_row.astype(jnp.int32)
        nm_ref[0] = jnp.where(rank_i < K, rank_i, -1)
        kk = lax.broadcasted_iota(jnp.int32, (K, N_PER), 0)
        Ob_f = jnp.where(rank_i == kk, 1.0, 0.0)
        iota_col = lax.broadcasted_iota(jnp.int32, (N_PER, 1), 0).astype(jnp.float32)
        order_col = _dot(Ob_f, iota_col, ((1,), (0,)))
        ik = lax.broadcasted_iota(jnp.int32, (K, K), 0)
        jk = lax.broadcasted_iota(jnp.int32, (K, K), 1)
        eye_k = jnp.where(ik == jk, 1.0, 0.0)
        order_row = _dot(order_col, eye_k, ((0,), (0,)))
        perm_ref[0] = order_row.astype(jnp.int32) + g * N_PER
        fp_ref[0] = _dot(Ob_f, fg, ((1,), (0,)))
        sl_col = _dot(fg, att_ref[0:1, :], ((1,), (1,)))
        sr_col = _dot(fg, att_ref[1:2, :], ((1,), (1,)))
        av_ref[0] = _dot(Ob_f, sl_col, ((1,), (0,)))
        b_col = _dot(Ob_f, sr_col, ((1,), (0,)))
        bv_ref[0] = _dot(b_col, eye_k, ((0,), (0,)))

    return pl.pallas_call(
        body,
        grid=(B,),
        in_specs=[
            pl.BlockSpec((1, N_PER, N_PER), lambda g: (g, 0, 0)),
            pl.BlockSpec((1, N_PER, D), lambda g: (g, 0, 0)),
            pl.BlockSpec((1, N_PER, 1), lambda g: (g, 0, 0)),
            pl.BlockSpec((1, N_PER, 1), lambda g: (g, 0, 0)),
            pl.BlockSpec((2, D), lambda g: (0, 0)),
        ],
        out_specs=[
            pl.BlockSpec((1, K, D), lambda g: (g, 0, 0)),
            pl.BlockSpec((1, 1, K), lambda g: (g, 0, 0)),
            pl.BlockSpec((1, 1, N_PER), lambda g: (g, 0, 0)),
            pl.BlockSpec((1, K, 1), lambda g: (g, 0, 0)),
            pl.BlockSpec((1, 1, K), lambda g: (g, 0, 0)),
        ],
        out_shape=[
            jax.ShapeDtypeStruct((B, K, D), jnp.float32),
            jax.ShapeDtypeStruct((B, 1, K), jnp.int32),
            jax.ShapeDtypeStruct((B, 1, N_PER), jnp.int32),
            jax.ShapeDtypeStruct((B, K, 1), jnp.float32),
            jax.ShapeDtypeStruct((B, 1, K), jnp.float32),
        ],
    )(A, feat_r, src_n, dst_n, att2)


def _tc_softmax(a3, b3, bias3):
    def body(a_ref, b_ref, bias_ref, ws_ref, row_ref, col_ref):
        g = pl.program_id(0)
        av = a_ref[0]
        bv = b_ref[0]
        w = av + bv
        w = jnp.where(w >= 0, w, SLOPE * w)
        w = w + bias_ref[0]
        m = jnp.max(w, axis=0, keepdims=True)
        ew = jnp.exp(w - m)
        den = jnp.sum(ew, axis=0, keepdims=True)
        ws_ref[0] = ew / den
        rr = lax.broadcasted_iota(jnp.int32, (K, K), 0)
        cc = lax.broadcasted_iota(jnp.int32, (K, K), 1)
        row_ref[0] = g * K + rr
        col_ref[0] = g * K + cc

    return pl.pallas_call(
        body,
        grid=(B,),
        in_specs=[
            pl.BlockSpec((1, K, 1), lambda g: (g, 0, 0)),
            pl.BlockSpec((1, 1, K), lambda g: (g, 0, 0)),
            pl.BlockSpec((1, K, K), lambda g: (g, 0, 0)),
        ],
        out_specs=[
            pl.BlockSpec((1, K, K), lambda g: (g, 0, 0)),
            pl.BlockSpec((1, K, K), lambda g: (g, 0, 0)),
            pl.BlockSpec((1, K, K), lambda g: (g, 0, 0)),
        ],
        out_shape=[
            jax.ShapeDtypeStruct((B, K, K), jnp.float32),
            jax.ShapeDtypeStruct((B, K, K), jnp.int32),
            jax.ShapeDtypeStruct((B, K, K), jnp.int32),
        ],
    )(a3, b3, bias3)


def kernel(feat, edge_index, e_feat, att):
    src = edge_index[0]
    dst = edge_index[1]
    A_flat, deg_flat = _sc_build_adj(src, dst, e_feat)
    A = A_flat.reshape(B, N_PER, N_PER)
    deg = deg_flat.reshape(B, 2, N_PER)
    src_norm = jnp.maximum(deg[:, 0], 1.0) ** -0.5
    dst_norm = jnp.maximum(deg[:, 1], 1.0) ** -0.5
    feat_r = feat.reshape(B, N_PER, D)
    att2 = att.reshape(2, D)
    feat_p, perm3, nm3, a3, b3 = _tc_score_topk(
        A, feat_r, src_norm.reshape(B, N_PER, 1), dst_norm.reshape(B, N_PER, 1), att2
    )
    bias_flat = _sc_build_bias(src, dst, e_feat, nm3.reshape(N))
    w3, row3, col3 = _tc_softmax(a3, b3, bias_flat.reshape(B, K, K))
    return (
        feat_p.reshape(PN, D),
        w3.reshape(NC),
        perm3.reshape(PN),
        row3.reshape(NC),
        col3.reshape(NC),
    )
